# Initial kernel scaffold; baseline (speedup 1.0000x reference)
import jax
import jax.numpy as jnp
from jax.experimental import pallas as pl


def kernel(positions, neighbor_matrix, num_neighbors):
    raise NotImplementedError("write your pallas kernel here")


# speedup vs baseline: 16.4888x; 16.4888x over previous
import functools

import jax
import jax.numpy as jnp
from jax import lax
from jax.experimental import pallas as pl
from jax.experimental.pallas import tpu as pltpu
from jax.experimental.pallas import tpu_sc as plsc

N_ATOMS = 100000
MAX_NEIGH = 32
NPAD = 102400
N_WORKERS = 32
ATOMS_PER_WORKER = NPAD // N_WORKERS
CHUNK = 128
N_CHUNKS = ATOMS_PER_WORKER // CHUNK
IDX_PER_CHUNK = CHUNK * MAX_NEIGH
GATHER_SPLIT = 128
N_GATHERS = IDX_PER_CHUNK // GATHER_SPLIT
CUTOFF2 = 36.0

_mesh = plsc.VectorSubcoreMesh(core_axis_name="c", subcore_axis_name="s")


@functools.partial(
    pl.kernel,
    mesh=_mesh,
    compiler_params=pltpu.CompilerParams(
        use_tc_tiling_on_sc=False, needs_layout_passes=False),
    out_type=(
        jax.ShapeDtypeStruct((NPAD, 4), jnp.float32),
        jax.ShapeDtypeStruct((N_WORKERS * 16,), jnp.float32),
    ),
    scratch_types=[
        pltpu.VMEM((N_GATHERS, GATHER_SPLIT), jnp.int32),
        pltpu.VMEM((IDX_PER_CHUNK, 8), jnp.float32),
        pltpu.VMEM((CHUNK, 8), jnp.float32),
        pltpu.VMEM((CHUNK,), jnp.int32),
        pltpu.VMEM((CHUNK, 4), jnp.float32),
        pltpu.VMEM((16,), jnp.float32),
        pltpu.SemaphoreType.DMA,
    ],
)
def _lj_sc(pos_hbm, idx_hbm, nn_hbm, fout_hbm, eout_hbm,
           idx_v, rows_v, own_v, nn_v, fout_v, e_v, sem):
    cid = lax.axis_index("c")
    sid = lax.axis_index("s")
    wid = sid * 2 + cid
    base = wid * ATOMS_PER_WORKER
    lanes = lax.iota(jnp.int32, 16)
    zeros = jnp.zeros((16,), jnp.float32)
    col0 = jnp.zeros((16,), jnp.int32)
    col1 = col0 + 1
    col2 = col0 + 2
    col3 = col0 + 3

    e_v[...] = zeros

    def chunk_body(c, _):
        row0 = base + c * CHUNK
        pltpu.sync_copy(
            idx_hbm.at[pl.ds(row0 * MAX_NEIGH // GATHER_SPLIT, N_GATHERS)],
            idx_v)
        pltpu.sync_copy(pos_hbm.at[pl.ds(row0, CHUNK)], own_v)
        pltpu.sync_copy(nn_hbm.at[pl.ds(row0, CHUNK)], nn_v)
        copies = []
        for j in range(N_GATHERS):
            sl = pl.ds(j * GATHER_SPLIT, GATHER_SPLIT)
            copies.append(
                pltpu.async_copy(pos_hbm.at[idx_v.at[j]], rows_v.at[sl], sem))
        for cp in copies:
            cp.wait()

        def i_body(i0, _):
            ai = i0 * 16 + lanes
            xi = plsc.load_gather(own_v, [ai, col0])
            yi = plsc.load_gather(own_v, [ai, col1])
            zi = plsc.load_gather(own_v, [ai, col2])
            nn16 = nn_v[pl.ds(i0 * 16, 16)]
            rbase = ai * MAX_NEIGH

            def m_body(m, carry):
                fx, fy, fz, e = carry
                r = rbase + m
                xj = plsc.load_gather(rows_v, [r, col0])
                yj = plsc.load_gather(rows_v, [r, col1])
                zj = plsc.load_gather(rows_v, [r, col2])
                dx = xj - xi
                dy = yj - yi
                dz = zj - zi
                r2 = dx * dx + dy * dy + dz * dz
                valid = (m < nn16) & (r2 < CUTOFF2) & (r2 > 1e-12)
                inv = 1.0 / jnp.where(valid, r2, 1.0)
                s6 = inv * inv * inv
                s12 = s6 * s6
                e = e + jnp.where(valid, s12 - s6, 0.0)
                fp = jnp.where(valid, (2.0 * s12 - s6) * inv, 0.0)
                return (fx + fp * dx, fy + fp * dy, fz + fp * dz, e)

            fx, fy, fz, e_out = lax.fori_loop(
                0, MAX_NEIGH, m_body, (zeros, zeros, zeros, zeros))
            plsc.store_scatter(fout_v, [ai, col0], -24.0 * fx)
            plsc.store_scatter(fout_v, [ai, col1], -24.0 * fy)
            plsc.store_scatter(fout_v, [ai, col2], -24.0 * fz)
            plsc.store_scatter(fout_v, [ai, col3], zeros)
            e_v[...] = e_v[...] + e_out
            return 0

        lax.fori_loop(0, CHUNK // 16, i_body, 0)
        pltpu.sync_copy(fout_v, fout_hbm.at[pl.ds(row0, CHUNK)])
        return 0

    lax.fori_loop(0, N_CHUNKS, chunk_body, 0)
    e_v[...] = 2.0 * e_v[...]
    pltpu.sync_copy(e_v, eout_hbm.at[pl.ds(wid * 16, 16)])


def kernel(positions, neighbor_matrix, num_neighbors):
    n, m = neighbor_matrix.shape
    pos8 = jnp.pad(positions.astype(jnp.float32), ((0, NPAD - n), (0, 5)))
    idx = jnp.pad(neighbor_matrix.astype(jnp.int32),
                  ((0, NPAD - n), (0, 0))).reshape(-1, GATHER_SPLIT)
    nn = jnp.pad(num_neighbors.astype(jnp.int32), (0, NPAD - n))
    fout, eout = _lj_sc(pos8, idx, nn)
    forces = fout[:n, :3]
    energies = jnp.sum(eout)[None]
    return energies, forces


# speedup vs baseline: 20.5618x; 1.2470x over previous
import functools

import jax
import jax.numpy as jnp
from jax import lax
from jax.experimental import pallas as pl
from jax.experimental.pallas import tpu as pltpu
from jax.experimental.pallas import tpu_sc as plsc

MAX_NEIGH = 32
NPAD = 102400
N_WORKERS = 32
ATOMS_PER_WORKER = NPAD // N_WORKERS
CHUNK = 128
N_CHUNKS = ATOMS_PER_WORKER // CHUNK
IDX_PER_CHUNK = CHUNK * MAX_NEIGH
GATHER_SPLIT = 128
N_GATHERS = IDX_PER_CHUNK // GATHER_SPLIT
CUTOFF2 = 36.0

_mesh = plsc.VectorSubcoreMesh(core_axis_name="c", subcore_axis_name="s")


@functools.partial(
    pl.kernel,
    mesh=_mesh,
    compiler_params=pltpu.CompilerParams(
        use_tc_tiling_on_sc=False, needs_layout_passes=False),
    out_type=(
        jax.ShapeDtypeStruct((NPAD, 4), jnp.float32),
        jax.ShapeDtypeStruct((N_WORKERS * 16,), jnp.float32),
    ),
    scratch_types=[
        pltpu.VMEM((N_GATHERS, GATHER_SPLIT), jnp.int32),
        pltpu.VMEM((N_GATHERS, GATHER_SPLIT), jnp.int32),
        pltpu.VMEM((IDX_PER_CHUNK, 8), jnp.float32),
        pltpu.VMEM((IDX_PER_CHUNK, 8), jnp.float32),
        pltpu.VMEM((CHUNK, 8), jnp.float32),
        pltpu.VMEM((CHUNK, 8), jnp.float32),
        pltpu.VMEM((CHUNK,), jnp.int32),
        pltpu.VMEM((CHUNK,), jnp.int32),
        pltpu.VMEM((CHUNK, 4), jnp.float32),
        pltpu.VMEM((CHUNK, 4), jnp.float32),
        pltpu.VMEM((16,), jnp.float32),
        pltpu.SemaphoreType.DMA,
        pltpu.SemaphoreType.DMA,
    ],
)
def _lj_sc(pos_hbm, idx_hbm, nn_hbm, fout_hbm, eout_hbm,
           idx_v0, idx_v1, rows_v0, rows_v1, own_v0, own_v1,
           nn_v0, nn_v1, fout_v0, fout_v1, e_v, sem0, sem1):
    cid = lax.axis_index("c")
    sid = lax.axis_index("s")
    wid = sid * 2 + cid
    base = wid * ATOMS_PER_WORKER
    lanes = lax.iota(jnp.int32, 16)
    zeros = jnp.zeros((16,), jnp.float32)
    col0 = jnp.zeros((16,), jnp.int32)
    col1 = col0 + 1
    col2 = col0 + 2
    col3 = col0 + 3
    bufs = ((idx_v0, rows_v0, own_v0, nn_v0, fout_v0, sem0),
            (idx_v1, rows_v1, own_v1, nn_v1, fout_v1, sem1))

    e_v[...] = zeros

    def stage(c, b, guarded):
        idx_v, rows_v, own_v, nn_v, _, sem = bufs[b]

        def do():
            row0 = base + c * CHUNK
            pltpu.sync_copy(
                idx_hbm.at[pl.ds(row0 * MAX_NEIGH // GATHER_SPLIT, N_GATHERS)],
                idx_v)
            pltpu.sync_copy(pos_hbm.at[pl.ds(row0, CHUNK)], own_v)
            pltpu.sync_copy(nn_hbm.at[pl.ds(row0, CHUNK)], nn_v)
            for j in range(N_GATHERS):
                sl = pl.ds(j * GATHER_SPLIT, GATHER_SPLIT)
                pltpu.async_copy(pos_hbm.at[idx_v.at[j]], rows_v.at[sl], sem)

        if guarded:
            pl.when(c < N_CHUNKS)(do)
        else:
            do()

    def compute(c, b):
        idx_v, rows_v, own_v, nn_v, fout_v, sem = bufs[b]
        for j in range(N_GATHERS):
            sl = pl.ds(j * GATHER_SPLIT, GATHER_SPLIT)
            pltpu.make_async_copy(pos_hbm.at[idx_v.at[j]],
                                  rows_v.at[sl], sem).wait()

        def i_body(i0, _):
            ai = i0 * 16 + lanes
            xi = plsc.load_gather(own_v, [ai, col0])
            yi = plsc.load_gather(own_v, [ai, col1])
            zi = plsc.load_gather(own_v, [ai, col2])
            nn16 = nn_v[pl.ds(i0 * 16, 16)]
            rbase = ai * MAX_NEIGH
            fx = fy = fz = e = zeros
            for m in range(MAX_NEIGH):
                r = rbase + m
                xj = plsc.load_gather(rows_v, [r, col0])
                yj = plsc.load_gather(rows_v, [r, col1])
                zj = plsc.load_gather(rows_v, [r, col2])
                dx = xj - xi
                dy = yj - yi
                dz = zj - zi
                r2 = dx * dx + dy * dy + dz * dz
                valid = (nn16 > m) & (r2 < CUTOFF2) & (r2 > 1e-12)
                inv = 1.0 / r2
                s6 = inv * inv * inv
                s12 = s6 * s6
                e = e + jnp.where(valid, s12 - s6, 0.0)
                fp = jnp.where(valid, (s12 + s12 - s6) * inv, 0.0)
                fx = fx + fp * dx
                fy = fy + fp * dy
                fz = fz + fp * dz
            plsc.store_scatter(fout_v, [ai, col0], -24.0 * fx)
            plsc.store_scatter(fout_v, [ai, col1], -24.0 * fy)
            plsc.store_scatter(fout_v, [ai, col2], -24.0 * fz)
            plsc.store_scatter(fout_v, [ai, col3], zeros)
            e_v[...] = e_v[...] + e
            return 0

        lax.fori_loop(0, CHUNK // 16, i_body, 0)
        pltpu.sync_copy(fout_v, fout_hbm.at[pl.ds(base + c * CHUNK, CHUNK)])

    stage(0, 0, False)
    stage(1, 1, False)

    def pair_body(k, _):
        c = k * 2
        compute(c, 0)
        stage(c + 2, 0, True)
        compute(c + 1, 1)
        stage(c + 3, 1, True)
        return 0

    lax.fori_loop(0, (N_CHUNKS - 1) // 2, pair_body, 0)
    compute(N_CHUNKS - 1, 0)

    e_v[...] = 2.0 * e_v[...]
    pltpu.sync_copy(e_v, eout_hbm.at[pl.ds(wid * 16, 16)])


def kernel(positions, neighbor_matrix, num_neighbors):
    n, m = neighbor_matrix.shape
    pos8 = jnp.pad(positions.astype(jnp.float32), ((0, NPAD - n), (0, 5)))
    idx = jnp.pad(neighbor_matrix.astype(jnp.int32),
                  ((0, NPAD - n), (0, 0))).reshape(-1, GATHER_SPLIT)
    nn = jnp.pad(num_neighbors.astype(jnp.int32), (0, NPAD - n))
    fout, eout = _lj_sc(pos8, idx, nn)
    forces = fout[:n, :3]
    energies = jnp.sum(eout)[None]
    return energies, forces


# speedup vs baseline: 33.7291x; 1.6404x over previous
import functools

import jax
import jax.numpy as jnp
from jax import lax
from jax.experimental import pallas as pl
from jax.experimental.pallas import tpu as pltpu
from jax.experimental.pallas import tpu_sc as plsc

N_ATOMS_C = 100000
MAX_NEIGH = 32
CHUNK = 80
N_CHUNKS_TOTAL = N_ATOMS_C // CHUNK
IDX_PER_CHUNK = CHUNK * MAX_NEIGH
GATHER_SPLIT = 128
N_GATHERS = IDX_PER_CHUNK // GATHER_SPLIT
IDX_ROWS = N_ATOMS_C * MAX_NEIGH // GATHER_SPLIT
CUTOFF2 = 36.0
BIG_WORKERS = N_CHUNKS_TOTAL - 32 * (N_CHUNKS_TOTAL // 32)
CHUNKS_SMALL = N_CHUNKS_TOTAL // 32

_mesh = plsc.VectorSubcoreMesh(core_axis_name="c", subcore_axis_name="s")


@functools.partial(
    pl.kernel,
    mesh=_mesh,
    compiler_params=pltpu.CompilerParams(
        use_tc_tiling_on_sc=False, needs_layout_passes=False),
    out_type=(
        jax.ShapeDtypeStruct((N_ATOMS_C * 3,), jnp.float32),
        jax.ShapeDtypeStruct((32 * 16,), jnp.float32),
    ),
    scratch_types=[
        pltpu.VMEM((N_GATHERS, GATHER_SPLIT), jnp.int32),
        pltpu.VMEM((N_GATHERS, GATHER_SPLIT), jnp.int32),
        pltpu.VMEM((IDX_PER_CHUNK, 8), jnp.float32),
        pltpu.VMEM((IDX_PER_CHUNK, 8), jnp.float32),
        pltpu.VMEM((CHUNK, 8), jnp.float32),
        pltpu.VMEM((CHUNK, 8), jnp.float32),
        pltpu.VMEM((CHUNK,), jnp.int32),
        pltpu.VMEM((CHUNK,), jnp.int32),
        pltpu.VMEM((CHUNK * 3,), jnp.float32),
        pltpu.VMEM((CHUNK * 3,), jnp.float32),
        pltpu.VMEM((16,), jnp.float32),
        pltpu.VMEM_SHARED((102400, 8), jnp.float32),
        pltpu.SemaphoreType.DMA,
        pltpu.SemaphoreType.DMA,
    ],
)
def _lj_sc(pos_hbm, idx_hbm, nn_hbm, fout_hbm, eout_hbm,
           idx_v0, idx_v1, rows_v0, rows_v1, own_v0, own_v1,
           nn_v0, nn_v1, fout_v0, fout_v1, e_v, pos_sh, sem0, sem1):
    cid = lax.axis_index("c")
    sid = lax.axis_index("s")
    wid = sid * 2 + cid
    cnt = jnp.where(wid < BIG_WORKERS, CHUNKS_SMALL + 1, CHUNKS_SMALL)
    start = wid * (CHUNKS_SMALL + 1) - jnp.maximum(wid - BIG_WORKERS, 0)
    lanes = lax.iota(jnp.int32, 16)
    zeros = jnp.zeros((16,), jnp.float32)
    col0 = jnp.zeros((16,), jnp.int32)
    col1 = col0 + 1
    col2 = col0 + 2
    bufs = ((idx_v0, rows_v0, own_v0, nn_v0, fout_v0, sem0),
            (idx_v1, rows_v1, own_v1, nn_v1, fout_v1, sem1))

    e_v[...] = zeros
    sh_rows = N_ATOMS_C // 16
    pltpu.sync_copy(pos_hbm.at[pl.ds(sid * sh_rows, sh_rows)],
                    pos_sh.at[pl.ds(sid * sh_rows, sh_rows)])
    plsc.subcore_barrier()

    def stage(c, b, guarded):
        idx_v, rows_v, own_v, nn_v, _, sem = bufs[b]

        def do():
            ch = start + c
            row0 = ch * CHUNK
            pltpu.sync_copy(idx_hbm.at[pl.ds(ch * N_GATHERS, N_GATHERS)],
                            idx_v)
            pltpu.sync_copy(pos_hbm.at[pl.ds(row0, CHUNK)], own_v)
            pltpu.sync_copy(nn_hbm.at[pl.ds(row0, CHUNK)], nn_v)
            for j in range(N_GATHERS):
                sl = pl.ds(j * GATHER_SPLIT, GATHER_SPLIT)
                pltpu.async_copy(pos_sh.at[idx_v.at[j]], rows_v.at[sl], sem)

        if guarded:
            pl.when(c < cnt)(do)
        else:
            do()

    def compute(c, b):
        idx_v, rows_v, own_v, nn_v, fout_v, sem = bufs[b]
        for j in range(N_GATHERS):
            sl = pl.ds(j * GATHER_SPLIT, GATHER_SPLIT)
            pltpu.make_async_copy(pos_sh.at[idx_v.at[j]],
                                  rows_v.at[sl], sem).wait()

        def i_body(i0, _):
            ai = i0 * 16 + lanes
            xi = plsc.load_gather(own_v, [ai, col0])
            yi = plsc.load_gather(own_v, [ai, col1])
            zi = plsc.load_gather(own_v, [ai, col2])
            nn16 = nn_v[pl.ds(i0 * 16, 16)]
            rbase = ai * MAX_NEIGH
            fx = fy = fz = e = zeros
            for m in range(MAX_NEIGH):
                r = rbase + m
                xj = plsc.load_gather(rows_v, [r, col0])
                yj = plsc.load_gather(rows_v, [r, col1])
                zj = plsc.load_gather(rows_v, [r, col2])
                dx = xj - xi
                dy = yj - yi
                dz = zj - zi
                r2 = dx * dx + dy * dy + dz * dz
                valid = (nn16 > m) & (r2 < CUTOFF2) & (r2 > 1e-12)
                inv = 1.0 / r2
                s6 = inv * inv * inv
                s12 = s6 * s6
                e = e + jnp.where(valid, s12 - s6, 0.0)
                fp = jnp.where(valid, (s12 + s12 - s6) * inv, 0.0)
                fx = fx + fp * dx
                fy = fy + fp * dy
                fz = fz + fp * dz
            a3 = ai * 3
            plsc.store_scatter(fout_v, [a3], -24.0 * fx)
            plsc.store_scatter(fout_v, [a3 + 1], -24.0 * fy)
            plsc.store_scatter(fout_v, [a3 + 2], -24.0 * fz)
            e_v[...] = e_v[...] + e
            return 0

        lax.fori_loop(0, CHUNK // 16, i_body, 0)
        pltpu.sync_copy(
            fout_v, fout_hbm.at[pl.ds((start + c) * CHUNK * 3, CHUNK * 3)])

    stage(0, 0, False)
    stage(1, 1, False)

    def pair_body(k, _):
        c = k * 2
        compute(c, 0)
        stage(c + 2, 0, True)
        compute(c + 1, 1)
        stage(c + 3, 1, True)
        return 0

    n_pairs = (cnt - 1) // 2
    lax.fori_loop(0, n_pairs, pair_body, 0)
    compute(2 * n_pairs, 0)
    pl.when(cnt - 2 * n_pairs == 2)(lambda: compute(2 * n_pairs + 1, 1))

    e_v[...] = 2.0 * e_v[...]
    pltpu.sync_copy(e_v, eout_hbm.at[pl.ds(wid * 16, 16)])


def kernel(positions, neighbor_matrix, num_neighbors):
    n, m = neighbor_matrix.shape
    pos8 = jnp.pad(positions.astype(jnp.float32), ((0, 0), (0, 5)))
    idx = neighbor_matrix.astype(jnp.int32).reshape(IDX_ROWS, GATHER_SPLIT)
    nn = num_neighbors.astype(jnp.int32)
    fout, eout = _lj_sc(pos8, idx, nn)
    forces = fout.reshape(n, 3)
    energies = jnp.sum(eout)[None]
    return energies, forces
